# baseline (speedup 1.0000x reference)
import functools

import jax
import jax.numpy as jnp
from jax import lax
from jax.experimental import pallas as pl
from jax.experimental.pallas import tpu as pltpu
from jax.experimental.pallas import tpu_sc as plsc

N = 10000
D = 128
E = 320000

NC = 2
NS = 16
NW = NC * NS
CHUNK = 64
EPW = 10240
EPAD = NW * EPW
NCH = EPW // CHUNK
SG = 4
SI = 8
SLAB = 640
LAST_SLAB = N - SLAB * (NS - 1)


def _sc_body(x_hbm, ed_hbm, vals_hbm, z_hbm, out_hbm,
             acc, eb, vbuf, gbuf, isem, vsem, gsem, ssem):
    c = lax.axis_index("c")
    s = lax.axis_index("s")
    wid = c * NS + s
    ebase = wid * EPW
    rbase = s * SLAB

    @pl.when(s < NS - 1)
    def _():
        pltpu.sync_copy(z_hbm, acc.at[pl.ds(rbase, SLAB)])

    @pl.when(s == NS - 1)
    def _():
        pltpu.sync_copy(z_hbm.at[pl.ds(0, LAST_SLAB)],
                        acc.at[pl.ds(rbase, LAST_SLAB)])

    plsc.subcore_barrier()

    def idx_refs(j):
        sl = lax.rem(j, SI)
        src = ed_hbm.at[:, pl.ds(ebase + j * CHUNK, CHUNK)]
        return src, eb.at[sl], isem.at[sl]

    def vals_refs(j):
        sl = lax.rem(j, SI)
        src = vals_hbm.at[pl.ds(ebase + j * CHUNK, CHUNK)]
        return src, vbuf.at[sl], vsem.at[sl]

    def gather_refs(j):
        sg = lax.rem(j, SG)
        si = lax.rem(j, SI)
        return x_hbm.at[eb.at[si].at[0]], gbuf.at[sg], gsem.at[sg]

    def scatter_refs(j):
        sg = lax.rem(j, SG)
        si = lax.rem(j, SI)
        return gbuf.at[sg], acc.at[eb.at[si].at[1]], ssem.at[sg]

    def idx_start(j):
        src, dst, sem = idx_refs(j)
        pltpu.async_copy(src, dst, sem)
        vsrc, vdst, vs = vals_refs(j)
        pltpu.async_copy(vsrc, vdst, vs)

    def idx_wait(j):
        src, dst, sem = idx_refs(j)
        pltpu.make_async_copy(src, dst, sem).wait()
        vsrc, vdst, vs = vals_refs(j)
        pltpu.make_async_copy(vsrc, vdst, vs).wait()

    def gather_start(j):
        src, dst, sem = gather_refs(j)
        pltpu.async_copy(src, dst, sem)

    def gather_wait(j):
        src, dst, sem = gather_refs(j)
        pltpu.make_async_copy(src, dst, sem).wait()

    def scatter_start(j):
        src, dst, sem = scatter_refs(j)
        pltpu.async_copy(src, dst, sem, add=True)

    def scatter_wait(j):
        src, dst, sem = scatter_refs(j)
        pltpu.make_async_copy(src, dst, sem).wait()

    idx_start(0)
    idx_start(1)
    idx_start(2)
    idx_wait(0)
    gather_start(0)
    idx_wait(1)
    gather_start(1)

    def body(j, carry):
        sg = lax.rem(j, SG)
        si = lax.rem(j, SI)
        gather_wait(j)

        gb = gbuf.at[sg]
        vbj = vbuf.at[si]
        for g in range(CHUNK // 16):
            vv = vbj[pl.ds(g * 16, 16)]
            for t in range(16):
                e = g * 16 + t
                vb = jnp.full((16,), vv[t], dtype=jnp.float32)
                for q in range(D // 16):
                    sl = pl.ds(q * 16, 16)
                    gb[e, sl] = gb[e, sl] * vb

        scatter_start(j)

        @pl.when(j >= 2)
        def _():
            scatter_wait(j - 2)

        @pl.when(j + 3 < NCH)
        def _():
            idx_start(j + 3)

        @pl.when(j + 2 < NCH)
        def _():
            idx_wait(j + 2)
            gather_start(j + 2)

        return carry

    lax.fori_loop(0, NCH, body, 0)
    scatter_wait(NCH - 2)
    scatter_wait(NCH - 1)

    plsc.subcore_barrier()

    @pl.when(s < NS - 1)
    def _():
        pltpu.sync_copy(acc.at[pl.ds(rbase, SLAB)],
                        out_hbm.at[c, pl.ds(rbase, SLAB)])

    @pl.when(s == NS - 1)
    def _():
        pltpu.sync_copy(acc.at[pl.ds(rbase, LAST_SLAB)],
                        out_hbm.at[c, pl.ds(rbase, LAST_SLAB)])


def _sc_segment_sum(x, packed, vals, zrows):
    mesh = plsc.VectorSubcoreMesh(core_axis_name="c", subcore_axis_name="s")
    fn = functools.partial(
        pl.kernel,
        out_type=jax.ShapeDtypeStruct((NC, N, D), jnp.float32),
        mesh=mesh,
        compiler_params=pltpu.CompilerParams(use_tc_tiling_on_sc=False),
        scratch_types=[
            pltpu.VMEM_SHARED((N, D), jnp.float32),
            pltpu.VMEM((SI, 2, CHUNK), jnp.int32),
            pltpu.VMEM((SI, CHUNK), jnp.float32),
            pltpu.VMEM((SG, CHUNK, D), jnp.float32),
            pltpu.SemaphoreType.DMA((SI,)),
            pltpu.SemaphoreType.DMA((SI,)),
            pltpu.SemaphoreType.DMA((SG,)),
            pltpu.SemaphoreType.DMA((SG,)),
        ],
    )(_sc_body)
    return fn(x, packed, vals, zrows)


BLK = 1000


def _tc_finalize(partial, W, b2):
    def body(p_ref, w_ref, b_ref, o_ref):
        s = p_ref[0] + p_ref[1]
        y = jnp.dot(s, w_ref[...], preferred_element_type=jnp.float32)
        o_ref[...] = jnp.maximum(y + b_ref[...], 0.0)

    return pl.pallas_call(
        body,
        grid=(N // BLK,),
        in_specs=[
            pl.BlockSpec((2, BLK, D), lambda i: (0, i, 0)),
            pl.BlockSpec((D, D), lambda i: (0, 0)),
            pl.BlockSpec((1, D), lambda i: (0, 0)),
        ],
        out_specs=pl.BlockSpec((BLK, D), lambda i: (i, 0)),
        out_shape=jax.ShapeDtypeStruct((N, D), jnp.float32),
    )(partial, W, b2)


def kernel(x, edge_index, edge_vals, W, b):
    pad = EPAD - E
    src = jnp.pad(edge_index[0].astype(jnp.int32), (0, pad))
    dst = jnp.pad(edge_index[1].astype(jnp.int32), (0, pad))
    vals_p = jnp.pad(edge_vals, (0, pad))
    packed = jnp.stack([src, dst])
    zrows = jnp.zeros((SLAB, D), jnp.float32)
    partial = _sc_segment_sum(x, packed, vals_p, zrows)
    return _tc_finalize(partial, W, b.reshape(1, D))
